# baseline (speedup 1.0000x reference)
import jax
import jax.numpy as jnp
from jax.experimental import pallas as pl
from jax.experimental.pallas import tpu as pltpu

_ROWS = 16384
_HID = 2048
_EXPERTS = 64
_BR = 1024
_NK = 2
_BK = _HID // _NK


def _router_kernel(x_ref, w_ref, val_ref, idx_ref, acc_ref):
    k = pl.program_id(1)
    part = jax.lax.dot_general(
        w_ref[...], x_ref[...], (((1,), (1,)), ((), ())),
        preferred_element_type=jnp.float32,
    )

    @pl.when(k == 0)
    def _():
        acc_ref[...] = part

    @pl.when(k > 0)
    def _():
        acc_ref[...] += part

    @pl.when(k == _NK - 1)
    def _():
        lg = acc_ref[...]
        iota = jax.lax.broadcasted_iota(jnp.int32, lg.shape, 0)
        m1 = jnp.max(lg, axis=0, keepdims=True)
        i1 = jnp.min(jnp.where(lg == m1, iota, _EXPERTS), axis=0, keepdims=True)
        masked = jnp.where(iota == i1, -jnp.inf, lg)
        m2 = jnp.max(masked, axis=0, keepdims=True)
        i2 = jnp.min(jnp.where(masked == m2, iota, _EXPERTS), axis=0, keepdims=True)
        e2 = jnp.exp(m2 - m1)
        inv = 1.0 / (1.0 + e2)
        val_ref[...] = jnp.concatenate([inv, e2 * inv], axis=0)
        idx_ref[...] = jnp.concatenate([i1, i2], axis=0)


@jax.jit
def kernel(hidden_states, weight):
    grid = (_ROWS // _BR, _NK)
    vals, idx = pl.pallas_call(
        _router_kernel,
        grid=grid,
        in_specs=[
            pl.BlockSpec((_BR, _BK), lambda i, k: (i, k)),
            pl.BlockSpec((_EXPERTS, _BK), lambda i, k: (0, k)),
        ],
        out_specs=[
            pl.BlockSpec((2, _BR), lambda i, k: (0, i)),
            pl.BlockSpec((2, _BR), lambda i, k: (0, i)),
        ],
        out_shape=[
            jax.ShapeDtypeStruct((2, _ROWS), jnp.float32),
            jax.ShapeDtypeStruct((2, _ROWS), jnp.int32),
        ],
        scratch_shapes=[pltpu.VMEM((_EXPERTS, _BR), jnp.float32)],
    )(hidden_states, weight)
    return (vals.T, idx.T)


# speedup vs baseline: 1.2760x; 1.2760x over previous
import jax
import jax.numpy as jnp
from jax.experimental import pallas as pl

_ROWS = 16384
_HID = 2048
_EXPERTS = 64
_BR = 1024


def _router_kernel(x_ref, w_ref, val_ref, idx_ref):
    lg = jax.lax.dot_general(
        w_ref[...], x_ref[...], (((1,), (1,)), ((), ())),
        preferred_element_type=jnp.float32,
    )
    iota = jax.lax.broadcasted_iota(jnp.int32, lg.shape, 0)
    m1 = jnp.max(lg, axis=0, keepdims=True)
    i1 = jnp.min(jnp.where(lg == m1, iota, _EXPERTS), axis=0, keepdims=True)
    masked = jnp.where(iota == i1, -jnp.inf, lg)
    m2 = jnp.max(masked, axis=0, keepdims=True)
    i2 = jnp.min(jnp.where(masked == m2, iota, _EXPERTS), axis=0, keepdims=True)
    e2 = jnp.exp(m2 - m1)
    inv = 1.0 / (1.0 + e2)
    val_ref[...] = jnp.concatenate([inv, e2 * inv], axis=0)
    idx_ref[...] = jnp.concatenate([i1, i2], axis=0)


@jax.jit
def kernel(hidden_states, weight):
    grid = (_ROWS // _BR,)
    vals, idx = pl.pallas_call(
        _router_kernel,
        grid=grid,
        in_specs=[
            pl.BlockSpec((_BR, _HID), lambda i: (i, 0)),
            pl.BlockSpec((_EXPERTS, _HID), lambda i: (0, 0)),
        ],
        out_specs=[
            pl.BlockSpec((2, _BR), lambda i: (0, i)),
            pl.BlockSpec((2, _BR), lambda i: (0, i)),
        ],
        out_shape=[
            jax.ShapeDtypeStruct((2, _ROWS), jnp.float32),
            jax.ShapeDtypeStruct((2, _ROWS), jnp.int32),
        ],
    )(hidden_states, weight)
    return (vals.T, idx.T)
